# baseline (speedup 1.0000x reference)
import functools

import numpy as np

import jax
import jax.numpy as jnp
from jax import lax
from jax.experimental import pallas as pl
from jax.experimental.pallas import tpu as pltpu
from jax.experimental.pallas import tpu_sc as plsc

BATCH = 64
NUM_PATCHES = 576
PATCH_DIM = 768
PROJ_DIM = 96
NUM_MASK = 432
NUM_UNMASK = 144

NW = 32
U_TOT = BATCH * NUM_UNMASK
U_HALF = U_TOT // 2
U_PER_W = U_HALF // NW
CHUNK = 72


def _threefry2x32(k1, k2, x0, x1):
    r0 = (13, 15, 26, 6)
    r1 = (17, 29, 16, 24)
    ks = (np.uint32(k1), np.uint32(k2),
          np.uint32(k1) ^ np.uint32(k2) ^ np.uint32(0x1BD11BDA))

    def rounds(x0, x1, rots):
        for r in rots:
            x0 = (x0 + x1).astype(np.uint32)
            x1 = (x1 << np.uint32(r)) | (x1 >> np.uint32(32 - r))
            x1 = x0 ^ x1
        return x0, x1

    with np.errstate(over="ignore"):
        x0 = (x0 + ks[0]).astype(np.uint32)
        x1 = (x1 + ks[1]).astype(np.uint32)
        for i, rots in enumerate((r0, r1, r0, r1, r0)):
            x0, x1 = rounds(x0, x1, rots)
            x0 = (x0 + ks[(i + 1) % 3]).astype(np.uint32)
            x1 = (x1 + ks[(i + 2) % 3] + np.uint32(i + 1)).astype(np.uint32)
    return x0, x1


def _masking_indices() -> np.ndarray:
    size = BATCH * NUM_PATCHES
    i64 = np.arange(size, dtype=np.uint64)
    c1 = (i64 >> np.uint64(32)).astype(np.uint32)
    c2 = (i64 & np.uint64(0xFFFFFFFF)).astype(np.uint32)
    b1, b2 = _threefry2x32(np.uint32(0), np.uint32(42), c1, c2)
    bits = (b1 ^ b2).reshape(BATCH, NUM_PATCHES)
    fb = (bits >> np.uint32(9)) | np.uint32(0x3F800000)
    u = np.maximum(np.float32(0), fb.view(np.float32) - np.float32(1.0))
    return np.argsort(u, axis=-1, kind="stable").astype(np.int32)


_RIDX = _masking_indices()
_MIDX = _RIDX[:, :NUM_MASK]
_UIDX = _RIDX[:, NUM_MASK:]
_UIDX_GLOB = np.ascontiguousarray(
    (_UIDX + np.arange(BATCH, dtype=np.int32)[:, None] * NUM_PATCHES)
    .reshape(-1))


def _sc_gather_body(patches_hbm, uidxg_hbm, g_out,
                    uidxg_v, prow0_v, prow1_v, gsem, ssem):
    wid = lax.axis_index("s") * 2 + lax.axis_index("c")
    ubase = wid * U_PER_W
    pltpu.sync_copy(uidxg_hbm.at[pl.ds(ubase, U_PER_W)], uidxg_v)
    pltpu.async_copy(
        patches_hbm.at[uidxg_v.at[pl.ds(0, CHUNK)]], prow0_v, gsem).wait()
    s0 = pltpu.async_copy(prow0_v, g_out.at[pl.ds(ubase, CHUNK)], ssem)
    pltpu.async_copy(
        patches_hbm.at[uidxg_v.at[pl.ds(CHUNK, CHUNK)]], prow1_v, gsem).wait()
    s1 = pltpu.async_copy(prow1_v, g_out.at[pl.ds(ubase + CHUNK, CHUNK)], ssem)
    s0.wait()
    s1.wait()


@functools.cache
def _sc_gather():
    mesh = plsc.VectorSubcoreMesh(core_axis_name="c", subcore_axis_name="s")
    return pl.kernel(
        _sc_gather_body,
        out_type=jax.ShapeDtypeStruct((U_HALF, PATCH_DIM), jnp.float32),
        mesh=mesh,
        scratch_types=[
            pltpu.VMEM((U_PER_W,), jnp.int32),
            pltpu.VMEM((CHUNK, PATCH_DIM), jnp.float32),
            pltpu.VMEM((CHUNK, PATCH_DIM), jnp.float32),
            pltpu.SemaphoreType.DMA,
            pltpu.SemaphoreType.DMA,
        ],
    )


def _pos_plus_t_body(mt_ref, w_ref, bt_ref, post_ref, out_ref):
    mtwt = lax.dot_general(
        w_ref[...], mt_ref[...], (((0,), (1,)), ((), ())))
    out_ref[...] = post_ref[...] + (mtwt + bt_ref[...])


_RB = 4
_NSTEP = BATCH // _RB
_PROJ_BLK = 768


def _onehot_body(ppt_ref, pos_ref, midx_ref, uidx_ref, mt_ref, up_ref):
    iota_m = lax.broadcasted_iota(jnp.int32, (NUM_PATCHES, NUM_MASK), 0)
    iota_u = lax.broadcasted_iota(jnp.int32, (NUM_UNMASK, NUM_PATCHES), 1)
    for r in range(_RB):
        oh_m = (iota_m == midx_ref[r]).astype(jnp.float32)
        mt_ref[r] = jnp.dot(ppt_ref[...], oh_m)
        oh_u = (iota_u == uidx_ref[r].reshape(NUM_UNMASK, 1)).astype(
            jnp.float32)
        up_ref[r] = jnp.dot(oh_u, pos_ref[...])


def _proj_body(x_ref, w_ref, b_ref, o_ref):
    o_ref[...] = jnp.dot(x_ref[...], w_ref[...]) + b_ref[...]


def kernel(patches, W, b, pos_table, mask_token):
    mask_indices = jnp.asarray(_MIDX)
    unmask_indices = jnp.asarray(_UIDX)
    b2 = b.reshape(1, PROJ_DIM)

    pos_plus_t = pl.pallas_call(
        _pos_plus_t_body,
        out_shape=jax.ShapeDtypeStruct((PROJ_DIM, NUM_PATCHES), jnp.float32),
    )(mask_token, W, b.reshape(PROJ_DIM, 1), pos_table.T)

    patches_flat = patches.reshape(BATCH * NUM_PATCHES, PATCH_DIM)
    g_rows_a = _sc_gather()(patches_flat, jnp.asarray(_UIDX_GLOB[:U_HALF]))
    g_rows_b = _sc_gather()(patches_flat, jnp.asarray(_UIDX_GLOB[U_HALF:]))

    mt, up = pl.pallas_call(
        _onehot_body,
        grid=(_NSTEP,),
        in_specs=[
            pl.BlockSpec((PROJ_DIM, NUM_PATCHES), lambda i: (0, 0)),
            pl.BlockSpec((NUM_PATCHES, PROJ_DIM), lambda i: (0, 0)),
            pl.BlockSpec((_RB, 1, NUM_MASK), lambda i: (i, 0, 0)),
            pl.BlockSpec((_RB, 1, NUM_UNMASK), lambda i: (i, 0, 0)),
        ],
        out_specs=[
            pl.BlockSpec((_RB, PROJ_DIM, NUM_MASK), lambda i: (i, 0, 0)),
            pl.BlockSpec((_RB, NUM_UNMASK, PROJ_DIM), lambda i: (i, 0, 0)),
        ],
        out_shape=[
            jax.ShapeDtypeStruct((BATCH, PROJ_DIM, NUM_MASK), jnp.float32),
            jax.ShapeDtypeStruct((BATCH, NUM_UNMASK, PROJ_DIM), jnp.float32),
        ],
    )(pos_plus_t, pos_table,
      jnp.asarray(_MIDX).reshape(BATCH, 1, NUM_MASK),
      jnp.asarray(_UIDX).reshape(BATCH, 1, NUM_UNMASK))

    def _proj_half(g_half):
        return pl.pallas_call(
            _proj_body,
            grid=(U_HALF // _PROJ_BLK,),
            in_specs=[
                pl.BlockSpec((_PROJ_BLK, PATCH_DIM), lambda i: (i, 0)),
                pl.BlockSpec((PATCH_DIM, PROJ_DIM), lambda i: (0, 0)),
                pl.BlockSpec((1, PROJ_DIM), lambda i: (0, 0)),
            ],
            out_specs=pl.BlockSpec((_PROJ_BLK, PROJ_DIM), lambda i: (i, 0)),
            out_shape=jax.ShapeDtypeStruct((U_HALF, PROJ_DIM), jnp.float32),
        )(g_half, W, b2)

    ue = jnp.concatenate([_proj_half(g_rows_a), _proj_half(g_rows_b)], axis=0)

    return (
        ue.reshape(BATCH, NUM_UNMASK, PROJ_DIM),
        jnp.transpose(mt, (0, 2, 1)),
        up,
        mask_indices,
        unmask_indices,
    )


# speedup vs baseline: 1.0911x; 1.0911x over previous
import functools

import numpy as np

import jax
import jax.numpy as jnp
from jax import lax
from jax.experimental import pallas as pl
from jax.experimental.pallas import tpu as pltpu
from jax.experimental.pallas import tpu_sc as plsc

BATCH = 64
NUM_PATCHES = 576
PATCH_DIM = 768
PROJ_DIM = 96
NUM_MASK = 432
NUM_UNMASK = 144

NW = 32
U_TOT = BATCH * NUM_UNMASK
U_PER_W = U_TOT // NW
CHUNK = 72


def _threefry2x32(k1, k2, x0, x1):
    r0 = (13, 15, 26, 6)
    r1 = (17, 29, 16, 24)
    ks = (np.uint32(k1), np.uint32(k2),
          np.uint32(k1) ^ np.uint32(k2) ^ np.uint32(0x1BD11BDA))

    def rounds(x0, x1, rots):
        for r in rots:
            x0 = (x0 + x1).astype(np.uint32)
            x1 = (x1 << np.uint32(r)) | (x1 >> np.uint32(32 - r))
            x1 = x0 ^ x1
        return x0, x1

    with np.errstate(over="ignore"):
        x0 = (x0 + ks[0]).astype(np.uint32)
        x1 = (x1 + ks[1]).astype(np.uint32)
        for i, rots in enumerate((r0, r1, r0, r1, r0)):
            x0, x1 = rounds(x0, x1, rots)
            x0 = (x0 + ks[(i + 1) % 3]).astype(np.uint32)
            x1 = (x1 + ks[(i + 2) % 3] + np.uint32(i + 1)).astype(np.uint32)
    return x0, x1


def _masking_indices() -> np.ndarray:
    size = BATCH * NUM_PATCHES
    i64 = np.arange(size, dtype=np.uint64)
    c1 = (i64 >> np.uint64(32)).astype(np.uint32)
    c2 = (i64 & np.uint64(0xFFFFFFFF)).astype(np.uint32)
    b1, b2 = _threefry2x32(np.uint32(0), np.uint32(42), c1, c2)
    bits = (b1 ^ b2).reshape(BATCH, NUM_PATCHES)
    fb = (bits >> np.uint32(9)) | np.uint32(0x3F800000)
    u = np.maximum(np.float32(0), fb.view(np.float32) - np.float32(1.0))
    return np.argsort(u, axis=-1, kind="stable").astype(np.int32)


_RIDX = _masking_indices()
_MIDX = _RIDX[:, :NUM_MASK]
_UIDX = _RIDX[:, NUM_MASK:]
_UIDX_GLOB = np.ascontiguousarray(
    (_UIDX + np.arange(BATCH, dtype=np.int32)[:, None] * NUM_PATCHES)
    .reshape(-1))


def _sc_gather_body(patches_hbm, uidxg_hbm, g_out,
                    uidxg_v, prow0_v, prow1_v, gsem, ssem):
    wid = lax.axis_index("s") * 2 + lax.axis_index("c")
    ubase = wid * U_PER_W
    pltpu.sync_copy(uidxg_hbm.at[pl.ds(ubase, U_PER_W)], uidxg_v)
    bufs = (prow0_v, prow1_v)
    nch = U_PER_W // CHUNK
    scat = [None, None]
    for c in range(nch):
        buf = bufs[c % 2]
        if scat[c % 2] is not None:
            scat[c % 2].wait()
        pltpu.async_copy(
            patches_hbm.at[uidxg_v.at[pl.ds(c * CHUNK, CHUNK)]], buf, gsem
        ).wait()
        scat[c % 2] = pltpu.async_copy(
            buf, g_out.at[pl.ds(ubase + c * CHUNK, CHUNK)], ssem)
    scat[0].wait()
    scat[1].wait()


@functools.cache
def _sc_gather():
    mesh = plsc.VectorSubcoreMesh(core_axis_name="c", subcore_axis_name="s")
    return pl.kernel(
        _sc_gather_body,
        out_type=jax.ShapeDtypeStruct((U_TOT, PATCH_DIM), jnp.float32),
        mesh=mesh,
        scratch_types=[
            pltpu.VMEM((U_PER_W,), jnp.int32),
            pltpu.VMEM((CHUNK, PATCH_DIM), jnp.float32),
            pltpu.VMEM((CHUNK, PATCH_DIM), jnp.float32),
            pltpu.SemaphoreType.DMA,
            pltpu.SemaphoreType.DMA,
        ],
    )


def _pos_plus_t_body(mt_ref, w_ref, bt_ref, post_ref, out_ref):
    mtwt = lax.dot_general(
        w_ref[...], mt_ref[...], (((0,), (1,)), ((), ())))
    out_ref[...] = post_ref[...] + (mtwt + bt_ref[...])


_RB = 4
_NSTEP = BATCH // _RB
_PROJ_BLK = 768


def _onehot_body(ppt_ref, pos_ref, midx_ref, uidx_ref, mt_ref, up_ref):
    iota_m = lax.broadcasted_iota(jnp.int32, (NUM_PATCHES, NUM_MASK), 0)
    iota_u = lax.broadcasted_iota(jnp.int32, (NUM_UNMASK, NUM_PATCHES), 1)
    for r in range(_RB):
        oh_m = (iota_m == midx_ref[r]).astype(jnp.float32)
        mt_ref[r] = jnp.dot(ppt_ref[...], oh_m)
        oh_u = (iota_u == uidx_ref[r].reshape(NUM_UNMASK, 1)).astype(
            jnp.float32)
        up_ref[r] = jnp.dot(oh_u, pos_ref[...])


def _proj_body(x_ref, w_ref, b_ref, o_ref):
    o_ref[...] = jnp.dot(x_ref[...], w_ref[...]) + b_ref[...]


def kernel(patches, W, b, pos_table, mask_token):
    mask_indices = jnp.asarray(_MIDX)
    unmask_indices = jnp.asarray(_UIDX)
    b2 = b.reshape(1, PROJ_DIM)

    pos_plus_t = pl.pallas_call(
        _pos_plus_t_body,
        out_shape=jax.ShapeDtypeStruct((PROJ_DIM, NUM_PATCHES), jnp.float32),
    )(mask_token, W, b.reshape(PROJ_DIM, 1), pos_table.T)

    g_rows = _sc_gather()(
        patches.reshape(BATCH * NUM_PATCHES, PATCH_DIM),
        jnp.asarray(_UIDX_GLOB))

    mt, up = pl.pallas_call(
        _onehot_body,
        grid=(_NSTEP,),
        in_specs=[
            pl.BlockSpec((PROJ_DIM, NUM_PATCHES), lambda i: (0, 0)),
            pl.BlockSpec((NUM_PATCHES, PROJ_DIM), lambda i: (0, 0)),
            pl.BlockSpec((_RB, 1, NUM_MASK), lambda i: (i, 0, 0)),
            pl.BlockSpec((_RB, 1, NUM_UNMASK), lambda i: (i, 0, 0)),
        ],
        out_specs=[
            pl.BlockSpec((_RB, PROJ_DIM, NUM_MASK), lambda i: (i, 0, 0)),
            pl.BlockSpec((_RB, NUM_UNMASK, PROJ_DIM), lambda i: (i, 0, 0)),
        ],
        out_shape=[
            jax.ShapeDtypeStruct((BATCH, PROJ_DIM, NUM_MASK), jnp.float32),
            jax.ShapeDtypeStruct((BATCH, NUM_UNMASK, PROJ_DIM), jnp.float32),
        ],
    )(pos_plus_t, pos_table,
      jnp.asarray(_MIDX).reshape(BATCH, 1, NUM_MASK),
      jnp.asarray(_UIDX).reshape(BATCH, 1, NUM_UNMASK))

    ue = pl.pallas_call(
        _proj_body,
        grid=(U_TOT // _PROJ_BLK,),
        in_specs=[
            pl.BlockSpec((_PROJ_BLK, PATCH_DIM), lambda i: (i, 0)),
            pl.BlockSpec((PATCH_DIM, PROJ_DIM), lambda i: (0, 0)),
            pl.BlockSpec((1, PROJ_DIM), lambda i: (0, 0)),
        ],
        out_specs=pl.BlockSpec((_PROJ_BLK, PROJ_DIM), lambda i: (i, 0)),
        out_shape=jax.ShapeDtypeStruct((U_TOT, PROJ_DIM), jnp.float32),
    )(g_rows, W, b2)

    return (
        ue.reshape(BATCH, NUM_UNMASK, PROJ_DIM),
        jnp.transpose(mt, (0, 2, 1)),
        up,
        mask_indices,
        unmask_indices,
    )


# speedup vs baseline: 1.1419x; 1.0466x over previous
import functools

import numpy as np

import jax
import jax.numpy as jnp
from jax import lax
from jax.experimental import pallas as pl
from jax.experimental.pallas import tpu as pltpu
from jax.experimental.pallas import tpu_sc as plsc

BATCH = 64
NUM_PATCHES = 576
PATCH_DIM = 768
PROJ_DIM = 96
NUM_MASK = 432
NUM_UNMASK = 144

NW = 32
U_TOT = BATCH * NUM_UNMASK
U_PER_W = U_TOT // NW
CHUNK = 96


def _threefry2x32(k1, k2, x0, x1):
    r0 = (13, 15, 26, 6)
    r1 = (17, 29, 16, 24)
    ks = (np.uint32(k1), np.uint32(k2),
          np.uint32(k1) ^ np.uint32(k2) ^ np.uint32(0x1BD11BDA))

    def rounds(x0, x1, rots):
        for r in rots:
            x0 = (x0 + x1).astype(np.uint32)
            x1 = (x1 << np.uint32(r)) | (x1 >> np.uint32(32 - r))
            x1 = x0 ^ x1
        return x0, x1

    with np.errstate(over="ignore"):
        x0 = (x0 + ks[0]).astype(np.uint32)
        x1 = (x1 + ks[1]).astype(np.uint32)
        for i, rots in enumerate((r0, r1, r0, r1, r0)):
            x0, x1 = rounds(x0, x1, rots)
            x0 = (x0 + ks[(i + 1) % 3]).astype(np.uint32)
            x1 = (x1 + ks[(i + 2) % 3] + np.uint32(i + 1)).astype(np.uint32)
    return x0, x1


def _masking_indices() -> np.ndarray:
    size = BATCH * NUM_PATCHES
    i64 = np.arange(size, dtype=np.uint64)
    c1 = (i64 >> np.uint64(32)).astype(np.uint32)
    c2 = (i64 & np.uint64(0xFFFFFFFF)).astype(np.uint32)
    b1, b2 = _threefry2x32(np.uint32(0), np.uint32(42), c1, c2)
    bits = (b1 ^ b2).reshape(BATCH, NUM_PATCHES)
    fb = (bits >> np.uint32(9)) | np.uint32(0x3F800000)
    u = np.maximum(np.float32(0), fb.view(np.float32) - np.float32(1.0))
    return np.argsort(u, axis=-1, kind="stable").astype(np.int32)


_RIDX = _masking_indices()
_MIDX = _RIDX[:, :NUM_MASK]
_UIDX = _RIDX[:, NUM_MASK:]
_UIDX_GLOB = np.ascontiguousarray(
    (_UIDX + np.arange(BATCH, dtype=np.int32)[:, None] * NUM_PATCHES)
    .reshape(-1))


def _sc_gather_body(patches_hbm, uidxg_hbm, g_out, uidxg_v, prow_v, sem):
    wid = lax.axis_index("s") * 2 + lax.axis_index("c")
    ubase = wid * U_PER_W
    pltpu.sync_copy(uidxg_hbm.at[pl.ds(ubase, U_PER_W)], uidxg_v)
    for c in range(U_PER_W // CHUNK):
        pltpu.async_copy(
            patches_hbm.at[uidxg_v.at[pl.ds(c * CHUNK, CHUNK)]], prow_v, sem
        ).wait()
        pltpu.sync_copy(prow_v, g_out.at[pl.ds(ubase + c * CHUNK, CHUNK)])


@functools.cache
def _sc_gather():
    mesh = plsc.VectorSubcoreMesh(core_axis_name="c", subcore_axis_name="s")
    return pl.kernel(
        _sc_gather_body,
        out_type=jax.ShapeDtypeStruct((U_TOT, PATCH_DIM), jnp.float32),
        mesh=mesh,
        scratch_types=[
            pltpu.VMEM((U_PER_W,), jnp.int32),
            pltpu.VMEM((CHUNK, PATCH_DIM), jnp.float32),
            pltpu.SemaphoreType.DMA,
        ],
    )


def _pos_plus_t_body(mt_ref, w_ref, bt_ref, post_ref, out_ref):
    mtwt = lax.dot_general(
        w_ref[...], mt_ref[...], (((0,), (1,)), ((), ())))
    out_ref[...] = post_ref[...] + (mtwt + bt_ref[...])


_RB = 8
_NSTEP = BATCH // _RB
_PROJ_BLK = 1536


def _onehot_body(ppt_ref, pos_ref, midx_ref, uidx_ref, mt_ref, up_ref):
    iota_m = lax.broadcasted_iota(jnp.int32, (NUM_PATCHES, NUM_MASK), 0)
    iota_u = lax.broadcasted_iota(jnp.int32, (NUM_UNMASK, NUM_PATCHES), 1)
    for r in range(_RB):
        oh_m = (iota_m == midx_ref[r]).astype(jnp.float32)
        mt_ref[r] = jnp.dot(ppt_ref[...], oh_m)
        oh_u = (iota_u == uidx_ref[r].reshape(NUM_UNMASK, 1)).astype(
            jnp.float32)
        up_ref[r] = jnp.dot(oh_u, pos_ref[...])


def _proj_body(x_ref, w_ref, b_ref, o_ref):
    o_ref[...] = jnp.dot(x_ref[...], w_ref[...]) + b_ref[...]


def kernel(patches, W, b, pos_table, mask_token):
    mask_indices = jnp.asarray(_MIDX)
    unmask_indices = jnp.asarray(_UIDX)
    b2 = b.reshape(1, PROJ_DIM)

    pos_plus_t = pl.pallas_call(
        _pos_plus_t_body,
        out_shape=jax.ShapeDtypeStruct((PROJ_DIM, NUM_PATCHES), jnp.float32),
    )(mask_token, W, b.reshape(PROJ_DIM, 1), pos_table.T)

    g_rows = _sc_gather()(
        patches.reshape(BATCH * NUM_PATCHES, PATCH_DIM),
        jnp.asarray(_UIDX_GLOB))

    mt, up = pl.pallas_call(
        _onehot_body,
        grid=(_NSTEP,),
        in_specs=[
            pl.BlockSpec((PROJ_DIM, NUM_PATCHES), lambda i: (0, 0)),
            pl.BlockSpec((NUM_PATCHES, PROJ_DIM), lambda i: (0, 0)),
            pl.BlockSpec((_RB, 1, NUM_MASK), lambda i: (i, 0, 0)),
            pl.BlockSpec((_RB, 1, NUM_UNMASK), lambda i: (i, 0, 0)),
        ],
        out_specs=[
            pl.BlockSpec((_RB, PROJ_DIM, NUM_MASK), lambda i: (i, 0, 0)),
            pl.BlockSpec((_RB, NUM_UNMASK, PROJ_DIM), lambda i: (i, 0, 0)),
        ],
        out_shape=[
            jax.ShapeDtypeStruct((BATCH, PROJ_DIM, NUM_MASK), jnp.float32),
            jax.ShapeDtypeStruct((BATCH, NUM_UNMASK, PROJ_DIM), jnp.float32),
        ],
    )(pos_plus_t, pos_table,
      jnp.asarray(_MIDX).reshape(BATCH, 1, NUM_MASK),
      jnp.asarray(_UIDX).reshape(BATCH, 1, NUM_UNMASK))

    ue = pl.pallas_call(
        _proj_body,
        grid=(U_TOT // _PROJ_BLK,),
        in_specs=[
            pl.BlockSpec((_PROJ_BLK, PATCH_DIM), lambda i: (i, 0)),
            pl.BlockSpec((PATCH_DIM, PROJ_DIM), lambda i: (0, 0)),
            pl.BlockSpec((1, PROJ_DIM), lambda i: (0, 0)),
        ],
        out_specs=pl.BlockSpec((_PROJ_BLK, PROJ_DIM), lambda i: (i, 0)),
        out_shape=jax.ShapeDtypeStruct((U_TOT, PROJ_DIM), jnp.float32),
    )(g_rows, W, b2)

    return (
        ue.reshape(BATCH, NUM_UNMASK, PROJ_DIM),
        jnp.transpose(mt, (0, 2, 1)),
        up,
        mask_indices,
        unmask_indices,
    )


# speedup vs baseline: 1.1859x; 1.0386x over previous
import functools

import numpy as np

import jax
import jax.numpy as jnp
from jax import lax
from jax.experimental import pallas as pl
from jax.experimental.pallas import tpu as pltpu
from jax.experimental.pallas import tpu_sc as plsc

BATCH = 64
NUM_PATCHES = 576
PATCH_DIM = 768
PROJ_DIM = 96
NUM_MASK = 432
NUM_UNMASK = 144

NW = 32
U_TOT = BATCH * NUM_UNMASK
U_PER_W = U_TOT // NW
CHUNK = 96


def _threefry2x32(k1, k2, x0, x1):
    r0 = (13, 15, 26, 6)
    r1 = (17, 29, 16, 24)
    ks = (np.uint32(k1), np.uint32(k2),
          np.uint32(k1) ^ np.uint32(k2) ^ np.uint32(0x1BD11BDA))

    def rounds(x0, x1, rots):
        for r in rots:
            x0 = (x0 + x1).astype(np.uint32)
            x1 = (x1 << np.uint32(r)) | (x1 >> np.uint32(32 - r))
            x1 = x0 ^ x1
        return x0, x1

    with np.errstate(over="ignore"):
        x0 = (x0 + ks[0]).astype(np.uint32)
        x1 = (x1 + ks[1]).astype(np.uint32)
        for i, rots in enumerate((r0, r1, r0, r1, r0)):
            x0, x1 = rounds(x0, x1, rots)
            x0 = (x0 + ks[(i + 1) % 3]).astype(np.uint32)
            x1 = (x1 + ks[(i + 2) % 3] + np.uint32(i + 1)).astype(np.uint32)
    return x0, x1


def _masking_indices() -> np.ndarray:
    size = BATCH * NUM_PATCHES
    i64 = np.arange(size, dtype=np.uint64)
    c1 = (i64 >> np.uint64(32)).astype(np.uint32)
    c2 = (i64 & np.uint64(0xFFFFFFFF)).astype(np.uint32)
    b1, b2 = _threefry2x32(np.uint32(0), np.uint32(42), c1, c2)
    bits = (b1 ^ b2).reshape(BATCH, NUM_PATCHES)
    fb = (bits >> np.uint32(9)) | np.uint32(0x3F800000)
    u = np.maximum(np.float32(0), fb.view(np.float32) - np.float32(1.0))
    return np.argsort(u, axis=-1, kind="stable").astype(np.int32)


_RIDX = _masking_indices()
_MIDX = _RIDX[:, :NUM_MASK]
_UIDX = _RIDX[:, NUM_MASK:]
_UIDX_GLOB = np.ascontiguousarray(
    (_UIDX + np.arange(BATCH, dtype=np.int32)[:, None] * NUM_PATCHES)
    .reshape(-1))


def _sc_gather_body(patches_hbm, uidxg_hbm, g_out, uidxg_v, prow_v, sem):
    wid = lax.axis_index("s") * 2 + lax.axis_index("c")
    ubase = wid * U_PER_W
    pltpu.sync_copy(uidxg_hbm.at[pl.ds(ubase, U_PER_W)], uidxg_v)
    for c in range(U_PER_W // CHUNK):
        pltpu.async_copy(
            patches_hbm.at[uidxg_v.at[pl.ds(c * CHUNK, CHUNK)]], prow_v, sem
        ).wait()
        pltpu.sync_copy(prow_v, g_out.at[pl.ds(ubase + c * CHUNK, CHUNK)])


@functools.cache
def _sc_gather():
    mesh = plsc.VectorSubcoreMesh(core_axis_name="c", subcore_axis_name="s")
    return pl.kernel(
        _sc_gather_body,
        out_type=jax.ShapeDtypeStruct((U_TOT, PATCH_DIM), jnp.float32),
        mesh=mesh,
        scratch_types=[
            pltpu.VMEM((U_PER_W,), jnp.int32),
            pltpu.VMEM((CHUNK, PATCH_DIM), jnp.float32),
            pltpu.SemaphoreType.DMA,
        ],
    )


def _pos_plus_t_body(mt_ref, w_ref, bt_ref, pos_ref, out_ref):
    mtwt = lax.dot_general(
        w_ref[...], mt_ref[...], (((0,), (1,)), ((), ())))
    out_ref[...] = jnp.transpose(pos_ref[...]) + (mtwt + bt_ref[...])


_RB = 8
_NSTEP = BATCH // _RB
_PROJ_BLK = 1536


def _onehot_body(ppt_ref, pos_ref, midx_ref, uidx_ref,
                 mt_ref, up_ref, mo_ref, uo_ref):
    iota_m = lax.broadcasted_iota(jnp.int32, (NUM_PATCHES, NUM_MASK), 0)
    iota_u = lax.broadcasted_iota(jnp.int32, (NUM_UNMASK, NUM_PATCHES), 1)
    for r in range(_RB):
        oh_m = (iota_m == midx_ref[r]).astype(jnp.float32)
        mt_ref[r] = jnp.dot(ppt_ref[...], oh_m)
        oh_u = (iota_u == uidx_ref[r].reshape(NUM_UNMASK, 1)).astype(
            jnp.float32)
        up_ref[r] = jnp.dot(oh_u, pos_ref[...])
    mo_ref[...] = midx_ref[...]
    uo_ref[...] = uidx_ref[...]


def _proj_body(x_ref, w_ref, b_ref, o_ref):
    o_ref[...] = jnp.dot(x_ref[...], w_ref[...]) + b_ref[...]


def kernel(patches, W, b, pos_table, mask_token):
    b2 = b.reshape(1, PROJ_DIM)

    pos_plus_t = pl.pallas_call(
        _pos_plus_t_body,
        out_shape=jax.ShapeDtypeStruct((PROJ_DIM, NUM_PATCHES), jnp.float32),
    )(mask_token, W, b.reshape(PROJ_DIM, 1), pos_table)

    g_rows = _sc_gather()(
        patches.reshape(BATCH * NUM_PATCHES, PATCH_DIM),
        jnp.asarray(_UIDX_GLOB))

    mt, up, mo, uo = pl.pallas_call(
        _onehot_body,
        grid=(_NSTEP,),
        in_specs=[
            pl.BlockSpec((PROJ_DIM, NUM_PATCHES), lambda i: (0, 0)),
            pl.BlockSpec((NUM_PATCHES, PROJ_DIM), lambda i: (0, 0)),
            pl.BlockSpec((_RB, 1, NUM_MASK), lambda i: (i, 0, 0)),
            pl.BlockSpec((_RB, 1, NUM_UNMASK), lambda i: (i, 0, 0)),
        ],
        out_specs=[
            pl.BlockSpec((_RB, PROJ_DIM, NUM_MASK), lambda i: (i, 0, 0)),
            pl.BlockSpec((_RB, NUM_UNMASK, PROJ_DIM), lambda i: (i, 0, 0)),
            pl.BlockSpec((_RB, 1, NUM_MASK), lambda i: (i, 0, 0)),
            pl.BlockSpec((_RB, 1, NUM_UNMASK), lambda i: (i, 0, 0)),
        ],
        out_shape=[
            jax.ShapeDtypeStruct((BATCH, PROJ_DIM, NUM_MASK), jnp.float32),
            jax.ShapeDtypeStruct((BATCH, NUM_UNMASK, PROJ_DIM), jnp.float32),
            jax.ShapeDtypeStruct((BATCH, 1, NUM_MASK), jnp.int32),
            jax.ShapeDtypeStruct((BATCH, 1, NUM_UNMASK), jnp.int32),
        ],
    )(pos_plus_t, pos_table,
      jnp.asarray(_MIDX).reshape(BATCH, 1, NUM_MASK),
      jnp.asarray(_UIDX).reshape(BATCH, 1, NUM_UNMASK))
    mask_indices = mo.reshape(BATCH, NUM_MASK)
    unmask_indices = uo.reshape(BATCH, NUM_UNMASK)

    ue = pl.pallas_call(
        _proj_body,
        grid=(U_TOT // _PROJ_BLK,),
        in_specs=[
            pl.BlockSpec((_PROJ_BLK, PATCH_DIM), lambda i: (i, 0)),
            pl.BlockSpec((PATCH_DIM, PROJ_DIM), lambda i: (0, 0)),
            pl.BlockSpec((1, PROJ_DIM), lambda i: (0, 0)),
        ],
        out_specs=pl.BlockSpec((_PROJ_BLK, PROJ_DIM), lambda i: (i, 0)),
        out_shape=jax.ShapeDtypeStruct((U_TOT, PROJ_DIM), jnp.float32),
    )(g_rows, W, b2)

    return (
        ue.reshape(BATCH, NUM_UNMASK, PROJ_DIM),
        jnp.transpose(mt, (0, 2, 1)),
        up,
        mask_indices,
        unmask_indices,
    )
